# baseline (speedup 1.0000x reference)
import functools

import jax
import jax.numpy as jnp
from jax import lax
from jax.experimental import pallas as pl
from jax.experimental.pallas import tpu as pltpu
from jax.experimental.pallas import tpu_sc as plsc

_BATCH = 16384
_HIST = 200
_EMBED = 16
_VOCAB = 1000000
_NC = 2
_NS = 16
_NW = _NC * _NS
_ROWS_PER_W = _BATCH // _NW
_CHUNK_ROWS = 16
_IDX_PER_CHUNK = _CHUNK_ROWS * _HIST
_STREAM = 128
_NSTREAM = _IDX_PER_CHUNK // _STREAM
_NCHUNK = _ROWS_PER_W // _CHUNK_ROWS

_TROWS = _VOCAB // 8
_TBLK = 1024
_TGRID = (_TROWS + _TBLK - 1) // _TBLK


def _proj_body(bs_ref, x_ref, s_ref, o_ref):
    o_ref[...] = (
        jax.lax.dot_general(
            s_ref[...], x_ref[...],
            dimension_numbers=(((0,), (1,)), ((), ())),
            preferred_element_type=jnp.float32)
        + bs_ref[0])


def _project(t128, S, bs):
    return pl.pallas_call(
        _proj_body,
        grid=(_TGRID,),
        in_specs=[
            pl.BlockSpec(memory_space=pltpu.SMEM),
            pl.BlockSpec((_TBLK, 128), lambda i: (i, 0)),
            pl.BlockSpec((128, 8), lambda i: (0, 0)),
        ],
        out_specs=pl.BlockSpec((8, _TBLK), lambda i: (0, i)),
        out_shape=jax.ShapeDtypeStruct((8, _TROWS), jnp.float32),
    )(bs, t128, S)


def _sc_body(xt_hbm, t_hbm, out_hbm, t_sh, idx_v, val_v, out_v, sem_i, sem_g):
    sid = lax.axis_index("s")
    wid = sid * _NC + lax.axis_index("c")

    stage = _VOCAB // 8

    @pl.when(sid < 8)
    def _():
        pltpu.sync_copy(t_hbm.at[pl.ds(sid * stage, stage)],
                        t_sh.at[pl.ds(sid * stage, stage)])
    plsc.subcore_barrier()

    base_idx = wid * (_ROWS_PER_W * _HIST)

    def chunk_body(c, carry):
        ioff = base_idx + c * _IDX_PER_CHUNK
        pltpu.async_copy(
            xt_hbm.at[pl.ds(ioff, _IDX_PER_CHUNK)], idx_v, sem_i).wait()
        for j in range(_NSTREAM):
            pltpu.async_copy(
                t_sh.at[idx_v.at[pl.ds(j * _STREAM, _STREAM)]],
                val_v.at[pl.ds(j * _STREAM, _STREAM)],
                sem_g)
        pltpu.make_async_copy(
            t_hbm.at[pl.ds(0, _IDX_PER_CHUNK)], val_v, sem_g).wait()

        def lbody(l, acc):
            return acc + val_v[pl.ds(l * 16, 16)]
        s = lax.fori_loop(0, _HIST, lbody, jnp.zeros((16,), jnp.float32))

        y = 1.0 / (1.0 + jnp.exp(-s))
        y = (y * 10000.0 + 0.5).astype(jnp.int32).astype(jnp.float32) * 1e-4
        out_v[pl.ds(c * _CHUNK_ROWS, _CHUNK_ROWS)] = y
        return carry

    lax.fori_loop(0, _NCHUNK, chunk_body, 0)
    pltpu.sync_copy(out_v, out_hbm.at[pl.ds(wid * _ROWS_PER_W, _ROWS_PER_W)])


@jax.jit
def _launch(xt, t128, S, bs):
    t8 = _project(t128, S, bs)
    t = t8.T.reshape(_VOCAB)
    mesh = plsc.VectorSubcoreMesh(core_axis_name="c", subcore_axis_name="s")
    f = functools.partial(
        pl.kernel,
        out_type=jax.ShapeDtypeStruct((_BATCH,), jnp.float32),
        mesh=mesh,
        compiler_params=pltpu.CompilerParams(use_tc_tiling_on_sc=False),
        scratch_types=[
            pltpu.VMEM_SHARED((_VOCAB,), jnp.float32),
            pltpu.VMEM((_IDX_PER_CHUNK,), jnp.int32),
            pltpu.VMEM((_IDX_PER_CHUNK,), jnp.float32),
            pltpu.VMEM((_ROWS_PER_W,), jnp.float32),
            pltpu.SemaphoreType.DMA,
            pltpu.SemaphoreType.DMA,
        ],
    )(_sc_body)
    return f(xt, t)


def kernel(x, table, W, b):
    xt = (x.astype(jnp.int32)
          .reshape(_BATCH // _CHUNK_ROWS, _CHUNK_ROWS, _HIST)
          .transpose(0, 2, 1)
          .reshape(_BATCH * _HIST))
    w = W.reshape(_EMBED).astype(jnp.float32) / float(_HIST)
    lanes = jnp.arange(128)
    S = jnp.where(lanes[:, None] // _EMBED == jnp.arange(8)[None, :],
                  jnp.tile(w, 8)[:, None], 0.0).astype(jnp.float32)
    bs = (b.astype(jnp.float32) / float(_HIST)).reshape(1)
    t128 = table.reshape(_TROWS, 128)
    out = _launch(xt, t128, S, bs)
    return out.reshape(_BATCH, 1)
